# baseline (speedup 1.0000x reference)
import jax
import jax.numpy as jnp
from jax.experimental import pallas as pl
from jax.experimental.pallas import tpu as pltpu

_BS = 512
_DB = 512


def _tc_body(start_ref, pe_ref, x_ref, o_ref):
    row0 = pl.multiple_of(start_ref[0], 8)
    pe_blk = pe_ref[pl.ds(row0, _BS), :]
    o_ref[...] = x_ref[...] + pe_blk[None, :, :]


@jax.jit
def _tc_pe_add(x, position_embedding, start):
    B, S, D = x.shape
    return pl.pallas_call(
        _tc_body,
        grid_spec=pltpu.PrefetchScalarGridSpec(
            num_scalar_prefetch=1,
            grid=(S // _BS, D // _DB),
            in_specs=[
                pl.BlockSpec((_BS, _DB), lambda i, j, s_ref: (i, j)),
                pl.BlockSpec((B, _BS, _DB), lambda i, j, s_ref: (0, i, j)),
            ],
            out_specs=pl.BlockSpec((B, _BS, _DB), lambda i, j, s_ref: (0, i, j)),
        ),
        out_shape=jax.ShapeDtypeStruct(x.shape, x.dtype),
        compiler_params=pltpu.CompilerParams(
            dimension_semantics=("parallel", "parallel"),
        ),
    )(start, position_embedding, x)


def kernel(x, position_embedding, position_start):
    start = jnp.asarray(position_start, jnp.int32).reshape((1,))
    return _tc_pe_add(x, position_embedding, start)


# speedup vs baseline: 1.0006x; 1.0006x over previous
import jax
import jax.numpy as jnp
from jax.experimental import pallas as pl
from jax.experimental.pallas import tpu as pltpu

_BS = 512


def _tc_body(start_ref, pe_ref, x_ref, o_ref):
    row0 = pl.multiple_of(start_ref[0], 8)
    pe_blk = pe_ref[pl.ds(row0, _BS), :]
    o_ref[...] = x_ref[...] + pe_blk[None, :, :]


@jax.jit
def _tc_pe_add(x, position_embedding, start):
    B, S, D = x.shape
    return pl.pallas_call(
        _tc_body,
        grid_spec=pltpu.PrefetchScalarGridSpec(
            num_scalar_prefetch=1,
            grid=(S // _BS,),
            in_specs=[
                pl.BlockSpec((_BS, D), lambda i, s_ref: (i, 0)),
                pl.BlockSpec((B, _BS, D), lambda i, s_ref: (0, i, 0)),
            ],
            out_specs=pl.BlockSpec((B, _BS, D), lambda i, s_ref: (0, i, 0)),
        ),
        out_shape=jax.ShapeDtypeStruct(x.shape, x.dtype),
        compiler_params=pltpu.CompilerParams(
            dimension_semantics=("parallel",),
        ),
    )(start, position_embedding, x)


def kernel(x, position_embedding, position_start):
    start = jnp.asarray(position_start, jnp.int32).reshape((1,))
    return _tc_pe_add(x, position_embedding, start)


# speedup vs baseline: 1.0632x; 1.0625x over previous
import jax
import jax.numpy as jnp
from jax.experimental import pallas as pl
from jax.experimental.pallas import tpu as pltpu


def _tc_body(start_ref, pe_ref, x_ref, o_ref):
    S = x_ref.shape[0]
    row0 = pl.multiple_of(start_ref[0], 8)
    o_ref[...] = x_ref[...] + pe_ref[pl.ds(row0, S), :]


import functools


@functools.partial(jax.jit, static_argnums=(3,))
def _tc_pe_add(x2d, position_embedding, start, batch):
    N, D = x2d.shape
    S = N // batch
    return pl.pallas_call(
        _tc_body,
        grid_spec=pltpu.PrefetchScalarGridSpec(
            num_scalar_prefetch=1,
            grid=(batch,),
            in_specs=[
                pl.BlockSpec(position_embedding.shape, lambda i, s_ref: (0, 0)),
                pl.BlockSpec((S, D), lambda i, s_ref: (i, 0)),
            ],
            out_specs=pl.BlockSpec((S, D), lambda i, s_ref: (i, 0)),
        ),
        out_shape=jax.ShapeDtypeStruct(x2d.shape, x2d.dtype),
        compiler_params=pltpu.CompilerParams(
            dimension_semantics=("parallel",),
        ),
    )(start, position_embedding, x2d)


def kernel(x, position_embedding, position_start):
    B, S, D = x.shape
    start = jnp.asarray(position_start, jnp.int32).reshape((1,))
    out2d = _tc_pe_add(x.reshape(B * S, D), position_embedding, start, B)
    return out2d.reshape(B, S, D)
